# baseline (speedup 1.0000x reference)
import functools

import jax
import jax.numpy as jnp
from jax import lax
from jax.experimental import pallas as pl
from jax.experimental.pallas import tpu as pltpu
from jax.experimental.pallas import tpu_sc as plsc

K = 1024
D = 256
N = 16384
NC = 512

NCORES = 2
NSUB = 16
NW = NCORES * NSUB
G = 8
COLS = D // G
TPG = NW // G
TPT = N // TPG
TCH = 256
NTCH = TPT // TCH
AW = K * COLS


def _assign_body(pn_ref, pp_ref, xn_ref, xx_ref, idx_ref, cnt_ref, acc_ref):
    i = pl.program_id(0)
    nblocks = pl.num_programs(0)

    @pl.when(i == 0)
    def _init():
        acc_ref[...] = jnp.zeros_like(acc_ref)

    pn = pn_ref[...]
    xn = xn_ref[...]
    pp = pp_ref[...]
    xx = xx_ref[...]

    s = lax.dot_general(pn, xn, (((1,), (1,)), ((), ())),
                        preferred_element_type=jnp.float32)
    d2 = (pp + xx) - 2.0 * s
    dist = jnp.sqrt(jnp.maximum(d2, 0.0))

    m = jnp.min(dist, axis=0, keepdims=True)
    iota_k = lax.broadcasted_iota(jnp.int32, (K, NC), 0)
    sel = jnp.where(dist == m, iota_k, K)
    idx = jnp.min(sel, axis=0, keepdims=True)
    idx_ref[...] = idx

    onehot = jnp.where(iota_k == idx, 1.0, 0.0)
    acc_ref[...] += jnp.sum(onehot, axis=1, keepdims=True)

    @pl.when(i == nblocks - 1)
    def _fin():
        cnt_ref[...] = acc_ref[...]


def _segsum_sc(xgt, idx, zeros):
    mesh = plsc.VectorSubcoreMesh(core_axis_name="c", subcore_axis_name="s")

    @functools.partial(
        pl.kernel,
        out_type=jax.ShapeDtypeStruct((TPG, G, AW), jnp.float32),
        mesh=mesh,
        scratch_types=[
            pltpu.VMEM((TCH,), jnp.int32),
            pltpu.VMEM((COLS, TCH), jnp.float32),
            pltpu.VMEM((AW,), jnp.float32),
        ],
        compiler_params=pltpu.CompilerParams(needs_layout_passes=False),
    )
    def scatter(xgt_hbm, idx_hbm, z_hbm, out_hbm, idx_vm, xbuf, acc):
        c = lax.axis_index("c")
        s = lax.axis_index("s")
        w = s * NCORES + c
        g = w // TPG
        p = w % TPG

        pltpu.sync_copy(z_hbm, acc)

        def _chunk(j, carry):
            t0 = p * TPT + j * TCH
            pltpu.sync_copy(idx_hbm.at[pl.ds(t0, TCH)], idx_vm)
            pltpu.sync_copy(xgt_hbm.at[g, :, pl.ds(t0, TCH)], xbuf)

            def _grp(tt, carry2):
                kkv = idx_vm[pl.ds(tt * 16, 16)]
                base = kkv * COLS
                for col in range(COLS):
                    xv = xbuf[col, pl.ds(tt * 16, 16)]
                    plsc.addupdate_scatter(acc, [base + col], xv)
                return carry2

            lax.fori_loop(0, TCH // 16, _grp, 0)
            return carry

        lax.fori_loop(0, NTCH, _chunk, 0)

        pltpu.sync_copy(acc, out_hbm.at[p, g])

    return scatter(xgt, idx, zeros)


def _fin_body(part_ref, cnt_ref, out_ref):
    sums = part_ref[0, 0]
    for q in range(1, TPG):
        sums = sums + part_ref[q, 0]
    cnts = cnt_ref[0]
    out_ref[...] = jnp.where(cnts > 0.0,
                             sums / jnp.maximum(cnts, 1.0),
                             jnp.zeros_like(sums))[None]


KR = AW // 128


@jax.jit
def _h2t(x, prototypes):
    pn = prototypes / jnp.linalg.norm(prototypes, axis=1)[:, None]
    xn = (x / jnp.linalg.norm(x, axis=-1)[..., None])[0]
    pp = jnp.sum(pn * pn, axis=1)[:, None]
    xx = jnp.sum(xn * xn, axis=1)[None, :]

    grid = N // NC
    idx, cnt = pl.pallas_call(
        _assign_body,
        grid=(grid,),
        in_specs=[
            pl.BlockSpec((K, D), lambda i: (0, 0)),
            pl.BlockSpec((K, 1), lambda i: (0, 0)),
            pl.BlockSpec((NC, D), lambda i: (i, 0)),
            pl.BlockSpec((1, NC), lambda i: (0, i)),
        ],
        out_specs=[
            pl.BlockSpec((1, NC), lambda i: (0, i)),
            pl.BlockSpec((K, 1), lambda i: (0, 0)),
        ],
        out_shape=[
            jax.ShapeDtypeStruct((1, N), jnp.int32),
            jax.ShapeDtypeStruct((K, 1), jnp.float32),
        ],
        scratch_shapes=[pltpu.VMEM((K, 1), jnp.float32)],
        compiler_params=pltpu.CompilerParams(
            dimension_semantics=("arbitrary",)),
    )(pn, pp, xn, xx)

    xgt = jnp.transpose(xn.reshape(N, G, COLS), (1, 2, 0))
    partials = _segsum_sc(xgt, idx.reshape(N),
                          jnp.zeros((AW,), jnp.float32))

    cntf = jnp.broadcast_to(cnt, (K, COLS)).reshape(1, KR, 128)
    partials4 = partials.reshape(TPG, G, KR, 128)

    means_flat = pl.pallas_call(
        _fin_body,
        grid=(G,),
        in_specs=[
            pl.BlockSpec((TPG, 1, KR, 128), lambda gi: (0, gi, 0, 0)),
            pl.BlockSpec((1, KR, 128), lambda gi: (0, 0, 0)),
        ],
        out_specs=pl.BlockSpec((1, KR, 128), lambda gi: (gi, 0, 0)),
        out_shape=jax.ShapeDtypeStruct((G, KR, 128), jnp.float32),
        compiler_params=pltpu.CompilerParams(
            dimension_semantics=("arbitrary",)),
    )(partials4, cntf)

    out = jnp.swapaxes(means_flat.reshape(G, K, COLS), 0, 1)
    return out.reshape(1, K * D)


def kernel(x, prototypes):
    return _h2t(x, prototypes)


# speedup vs baseline: 2.7405x; 2.7405x over previous
import functools

import jax
import jax.numpy as jnp
from jax import lax
from jax.experimental import pallas as pl
from jax.experimental.pallas import tpu as pltpu

K = 1024
D = 256
N = 16384
NC = 512


def _body(pn_ref, pp_ref, xn_ref, xx_ref, out_ref, sums_ref, cnts_ref):
    i = pl.program_id(0)
    nblocks = pl.num_programs(0)

    @pl.when(i == 0)
    def _init():
        sums_ref[...] = jnp.zeros_like(sums_ref)
        cnts_ref[...] = jnp.zeros_like(cnts_ref)

    pn = pn_ref[...]
    xn = xn_ref[...]
    pp = pp_ref[...]
    xx = xx_ref[...]

    s = lax.dot_general(pn, xn, (((1,), (1,)), ((), ())),
                        preferred_element_type=jnp.float32)
    d2 = (pp + xx) - 2.0 * s
    dist = jnp.sqrt(jnp.maximum(d2, 0.0))

    m = jnp.min(dist, axis=0, keepdims=True)
    iota_k = lax.broadcasted_iota(jnp.int32, (K, NC), 0)
    sel = jnp.where(dist == m, iota_k, K)
    idx = jnp.min(sel, axis=0, keepdims=True)

    onehot = jnp.where(iota_k == idx, 1.0, 0.0)

    sums_ref[...] += lax.dot_general(
        onehot, xn, (((1,), (0,)), ((), ())),
        preferred_element_type=jnp.float32)
    cnts_ref[...] += jnp.sum(onehot, axis=1, keepdims=True)

    @pl.when(i == nblocks - 1)
    def _fin():
        cnts = cnts_ref[...]
        sums = sums_ref[...]
        out_ref[...] = jnp.where(cnts > 0.0,
                                 sums / jnp.maximum(cnts, 1.0),
                                 jnp.zeros_like(sums))


@functools.partial(jax.jit, static_argnames=("interpret",))
def _h2t(x, prototypes, interpret=False):
    pn = prototypes / jnp.linalg.norm(prototypes, axis=1)[:, None]
    xn = (x / jnp.linalg.norm(x, axis=-1)[..., None])[0]
    pp = jnp.sum(pn * pn, axis=1)[:, None]
    xx = jnp.sum(xn * xn, axis=1)[None, :]

    grid = N // NC
    out = pl.pallas_call(
        _body,
        grid=(grid,),
        in_specs=[
            pl.BlockSpec((K, D), lambda i: (0, 0)),
            pl.BlockSpec((K, 1), lambda i: (0, 0)),
            pl.BlockSpec((NC, D), lambda i: (i, 0)),
            pl.BlockSpec((1, NC), lambda i: (0, i)),
        ],
        out_specs=pl.BlockSpec((K, D), lambda i: (0, 0)),
        out_shape=jax.ShapeDtypeStruct((K, D), jnp.float32),
        scratch_shapes=[
            pltpu.VMEM((K, D), jnp.float32),
            pltpu.VMEM((K, 1), jnp.float32),
        ],
        compiler_params=pltpu.CompilerParams(
            dimension_semantics=("arbitrary",)),
        interpret=interpret,
    )(pn, pp, xn, xx)
    return out.reshape(1, K * D)


def kernel(x, prototypes):
    return _h2t(x, prototypes)
